# baseline (speedup 1.0000x reference)
import functools

import jax
import jax.numpy as jnp
from jax import lax
from jax.experimental import pallas as pl
from jax.experimental.pallas import tpu as pltpu
from jax.experimental.pallas import tpu_sc as plsc

E = 3_200_000
DE = 16
DG = 128
NG = 256

_TILE = 128
_NT = E // _TILE
_CT = 16
_CE = _CT * _TILE
_HALF = _NT * 8 * _TILE


def _seg_reduce_sc(fe2, seg_ids):
    info = plsc.get_sparse_core_info()
    nc, ns = info.num_cores, info.num_subcores
    nw = nc * ns
    base_t = _NT // nw
    rem_t = _NT % nw
    n_chunks = base_t // _CT + 1
    mesh = plsc.VectorSubcoreMesh(core_axis_name="c", subcore_axis_name="s")

    neg_inf = jnp.float32(-jnp.inf)
    pos_inf = jnp.float32(jnp.inf)

    @functools.partial(
        pl.kernel,
        mesh=mesh,
        compiler_params=pltpu.CompilerParams(
            use_tc_tiling_on_sc=False, needs_layout_passes=False),
        out_type=[
            jax.ShapeDtypeStruct((nw * DE, NG), jnp.float32),
            jax.ShapeDtypeStruct((nw * DE, NG), jnp.float32),
            jax.ShapeDtypeStruct((nw * DE, NG), jnp.float32),
            jax.ShapeDtypeStruct((nw * DE, NG), jnp.float32),
        ],
        scratch_types=[
            pltpu.VMEM((2 * _CE * 8,), jnp.float32),
            pltpu.VMEM((2 * _CE * 8,), jnp.float32),
            pltpu.VMEM((_CE,), jnp.int32),
            pltpu.VMEM((_CE,), jnp.int32),
            pltpu.VMEM((DE, NG), jnp.float32),
            pltpu.VMEM((DE, NG), jnp.float32),
            pltpu.VMEM((DE, NG), jnp.float32),
            pltpu.VMEM((DE, NG), jnp.float32),
            pltpu.VMEM((DE * 16,), jnp.float32),
            pltpu.VMEM((DE * 16,), jnp.float32),
            pltpu.VMEM((DE * 16,), jnp.float32),
            pltpu.VMEM((DE,), jnp.float32),
            pltpu.VMEM((DE,), jnp.float32),
            pltpu.VMEM((DE,), jnp.float32),
            pltpu.SMEM((1,), jnp.int32),
            pltpu.SMEM((1,), jnp.float32),
            pltpu.SemaphoreType.DMA,
            pltpu.SemaphoreType.DMA,
            pltpu.SemaphoreType.DMA,
            pltpu.SemaphoreType.DMA,
        ],
    )
    def k(fe_hbm, ids_hbm, o_sum, o_min, o_max, o_cnt,
          fe_v0, fe_v1, ids_v0, ids_v1, sum_t, min_t, max_t, cnt_t,
          es_s, es_mn, es_mx, fs_s, fs_mn, fs_mx, cur_ref, cnt_ref,
          fsem0, fsem1, isem0, isem1):
        wid = lax.axis_index("c") * ns + lax.axis_index("s")
        tb = wid * base_t + jnp.minimum(wid, rem_t)
        ntiles = base_t + jnp.where(wid < rem_t, 1, 0)
        tail_t0 = tb + ntiles - _CT
        tail_skip = _CT - (ntiles - (n_chunks - 1) * _CT)

        fe_bufs = (fe_v0, fe_v1)
        ids_bufs = (ids_v0, ids_v1)
        fsems = (fsem0, fsem1)
        isems = (isem0, isem1)

        dvec = lax.iota(jnp.int32, 16)
        featoff = (dvec >> 3) * (_CE * 8) + (dvec & 7) * _TILE

        def tile0_of(c):
            return jnp.where(c < n_chunks - 1, tb + c * _CT, tail_t0)

        def fe_dma(c, b, half):
            start = tile0_of(c) * (8 * _TILE)
            return pltpu.make_async_copy(
                fe_hbm.at[half, pl.ds(start, _CE * 8)],
                fe_bufs[b].at[pl.ds(half * (_CE * 8), _CE * 8)], fsems[b])

        def ids_dma(c, b):
            start = tile0_of(c) * _TILE
            return pltpu.make_async_copy(
                ids_hbm.at[pl.ds(start, _CE)], ids_bufs[b], isems[b])

        def init_col(j, _):
            z16 = jnp.zeros((16,), jnp.float32)
            p16 = jnp.full((16,), pos_inf, jnp.float32)
            n16 = jnp.full((16,), neg_inf, jnp.float32)
            for d in range(DE):
                sum_t[d, pl.ds(j * 16, 16)] = z16
                min_t[d, pl.ds(j * 16, 16)] = p16
                max_t[d, pl.ds(j * 16, 16)] = n16
                cnt_t[d, pl.ds(j * 16, 16)] = z16
            return 0

        lax.fori_loop(0, NG // 16, init_col, 0)

        def reset_accs():
            for d in range(DE):
                es_s[pl.ds(d * 16, 16)] = jnp.zeros((16,), jnp.float32)
                es_mn[pl.ds(d * 16, 16)] = jnp.full((16,), pos_inf,
                                                    jnp.float32)
                es_mx[pl.ds(d * 16, 16)] = jnp.full((16,), neg_inf,
                                                    jnp.float32)
            fs_s[...] = jnp.zeros((DE,), jnp.float32)
            fs_mn[...] = jnp.full((DE,), pos_inf, jnp.float32)
            fs_mx[...] = jnp.full((DE,), neg_inf, jnp.float32)
            cnt_ref[0] = jnp.float32(0.0)

        reset_accs()
        cur_ref[0] = jnp.int32(-1)

        def flush():
            cur = cur_ref[0]

            @pl.when(cur >= 0)
            def _():
                row_s = fs_s[...]
                row_mn = fs_mn[...]
                row_mx = fs_mx[...]
                for d in range(DE):
                    oh = dvec == d
                    vs = jnp.sum(es_s[pl.ds(d * 16, 16)])
                    vmn = jnp.min(es_mn[pl.ds(d * 16, 16)])
                    vmx = jnp.max(es_mx[pl.ds(d * 16, 16)])
                    row_s = row_s + jnp.where(
                        oh, jax.lax.broadcast(vs, (16,)), 0.0)
                    row_mn = jnp.minimum(row_mn, jnp.where(
                        oh, jax.lax.broadcast(vmn, (16,)), pos_inf))
                    row_mx = jnp.maximum(row_mx, jnp.where(
                        oh, jax.lax.broadcast(vmx, (16,)), neg_inf))
                curv = jax.lax.broadcast(cur, (16,))
                old_s = plsc.load_gather(sum_t, [dvec, curv])
                plsc.store_scatter(sum_t, [dvec, curv], old_s + row_s)
                old_mn = plsc.load_gather(min_t, [dvec, curv])
                plsc.store_scatter(min_t, [dvec, curv],
                                   jnp.minimum(old_mn, row_mn))
                old_mx = plsc.load_gather(max_t, [dvec, curv])
                plsc.store_scatter(max_t, [dvec, curv],
                                   jnp.maximum(old_mx, row_mx))
                old_c = plsc.load_gather(cnt_t, [dvec, curv])
                plsc.store_scatter(cnt_t, [dvec, curv],
                                   old_c + jnp.full((16,), cnt_ref[0]))
                reset_accs()

        def process_chunk(fe_b, ids_b, t_lo):
            g_lo = t_lo * 8
            c_first = ids_b[pl.ds(t_lo * _TILE, 16)][0]
            c_last = ids_b[pl.ds(_CE - 16, 16)][15]
            ccur = cur_ref[0]
            cfast = jnp.logical_and(c_first == ccur, c_last == ccur)

            @pl.when(cfast)
            def _():
                for pf in range(2):
                    accs = []
                    for j in range(8):
                        o = (pf * 8 + j) * 16
                        accs.append(es_s[pl.ds(o, 16)])
                        accs.append(es_mn[pl.ds(o, 16)])
                        accs.append(es_mx[pl.ds(o, 16)])

                    def pass_body(t, carry, pf=pf):
                        tbase = t * (8 * _TILE)
                        out = list(carry)
                        for g8 in range(8):
                            for j in range(8):
                                off = (pf * (_CE * 8) + j * _TILE
                                       + g8 * 16) + tbase
                                v = fe_b[pl.ds(off, 16)]
                                out[3 * j] = out[3 * j] + v
                                out[3 * j + 1] = jnp.minimum(
                                    out[3 * j + 1], v)
                                out[3 * j + 2] = jnp.maximum(
                                    out[3 * j + 2], v)
                        return tuple(out)

                    accs = lax.fori_loop(t_lo, _CT, pass_body, tuple(accs))
                    for j in range(8):
                        o = (pf * 8 + j) * 16
                        es_s[pl.ds(o, 16)] = accs[3 * j]
                        es_mn[pl.ds(o, 16)] = accs[3 * j + 1]
                        es_mx[pl.ds(o, 16)] = accs[3 * j + 2]
                cnt_ref[0] = cnt_ref[0] + (
                    jnp.float32(_CE) - jnp.float32(16.0) * g_lo.astype(
                        jnp.float32))

            @pl.when(jnp.logical_not(cfast))
            def _():
                _process_tiles(fe_b, ids_b, t_lo)

        def _process_tiles(fe_b, ids_b, t_lo):
            def tile_body(t, _):
                e0 = t * _TILE
                id_a = ids_b[pl.ds(e0, 16)][0]
                id_b = ids_b[pl.ds(e0 + 112, 16)][15]
                cur = cur_ref[0]
                tfast = jnp.logical_and(id_a == cur, id_b == cur)

                @pl.when(tfast)
                def _():
                    for pf in range(2):
                        accs = []
                        for j in range(8):
                            d = pf * 8 + j
                            o = d * 16
                            accs.append([es_s[pl.ds(o, 16)],
                                         es_mn[pl.ds(o, 16)],
                                         es_mx[pl.ds(o, 16)]])
                        for g in range(8):
                            for j in range(8):
                                d = pf * 8 + j
                                off = ((d >> 3) * (_CE * 8) + (d & 7) * _TILE
                                       + g * 16) + t * (8 * _TILE)
                                v = fe_b[pl.ds(off, 16)]
                                a = accs[j]
                                a[0] = a[0] + v
                                a[1] = jnp.minimum(a[1], v)
                                a[2] = jnp.maximum(a[2], v)
                        for j in range(8):
                            d = pf * 8 + j
                            o = d * 16
                            es_s[pl.ds(o, 16)] = accs[j][0]
                            es_mn[pl.ds(o, 16)] = accs[j][1]
                            es_mx[pl.ds(o, 16)] = accs[j][2]
                    cnt_ref[0] = cnt_ref[0] + jnp.float32(_TILE)

                @pl.when(jnp.logical_not(tfast))
                def _():
                    def group_body(g, _g):
                        idv = ids_b[pl.ds(e0 + g * 16, 16)]
                        curg = cur_ref[0]
                        gfast = jnp.logical_and(idv[0] == curg,
                                                idv[15] == curg)

                        @pl.when(gfast)
                        def _():
                            for d in range(DE):
                                off = ((d >> 3) * (_CE * 8)
                                       + (d & 7) * _TILE) + t * (8 * _TILE)
                                v = fe_b[pl.ds(off + g * 16, 16)]
                                o = d * 16
                                es_s[pl.ds(o, 16)] = es_s[pl.ds(o, 16)] + v
                                es_mn[pl.ds(o, 16)] = jnp.minimum(
                                    es_mn[pl.ds(o, 16)], v)
                                es_mx[pl.ds(o, 16)] = jnp.maximum(
                                    es_mx[pl.ds(o, 16)], v)
                            cnt_ref[0] = cnt_ref[0] + jnp.float32(16.0)

                        @pl.when(jnp.logical_not(gfast))
                        def _():
                            for l in range(16):
                                eid = idv[l]

                                @pl.when(eid != cur_ref[0])
                                def _(eid=eid):
                                    flush()
                                    cur_ref[0] = eid

                                idx = featoff + (t * (8 * _TILE)
                                                 + g * 16 + l)
                                row = plsc.load_gather(fe_b, [idx])
                                fs_s[...] = fs_s[...] + row
                                fs_mn[...] = jnp.minimum(fs_mn[...], row)
                                fs_mx[...] = jnp.maximum(fs_mx[...], row)
                                cnt_ref[0] = cnt_ref[0] + 1.0

                        return 0

                    lax.fori_loop(0, 8, group_body, 0)

                return 0

            lax.fori_loop(t_lo, _CT, tile_body, 0)

        fe_dma(0, 0, 0).start()
        fe_dma(0, 0, 1).start()
        ids_dma(0, 0).start()

        def pair_body(i, _):
            for b in range(2):
                c = i * 2 + b

                @pl.when(c < n_chunks)
                def _(c=c, b=b):
                    @pl.when(c + 1 < n_chunks)
                    def _():
                        fe_dma(c + 1, 1 - b, 0).start()
                        fe_dma(c + 1, 1 - b, 1).start()
                        ids_dma(c + 1, 1 - b).start()

                    fe_dma(c, b, 0).wait()
                    fe_dma(c, b, 1).wait()
                    ids_dma(c, b).wait()
                    t_lo = jnp.where(c < n_chunks - 1, 0, tail_skip)
                    process_chunk(fe_bufs[b], ids_bufs[b], t_lo)
            return 0

        lax.fori_loop(0, (n_chunks + 1) // 2, pair_body, 0)
        flush()

        pltpu.sync_copy(sum_t, o_sum.at[pl.ds(wid * DE, DE), :])
        pltpu.sync_copy(min_t, o_min.at[pl.ds(wid * DE, DE), :])
        pltpu.sync_copy(max_t, o_max.at[pl.ds(wid * DE, DE), :])
        pltpu.sync_copy(cnt_t, o_cnt.at[pl.ds(wid * DE, DE), :])

    return k(fe2, seg_ids)


def _merge_tc_body(ps_ref, pm_ref, px_ref, pc_ref, wt_ref, b_ref, out_ref):
    s = jnp.sum(ps_ref[...], axis=0)
    mn = jnp.min(pm_ref[...], axis=0)
    mx = jnp.max(px_ref[...], axis=0)
    c = jnp.sum(pc_ref[...], axis=0)
    mean = s / jnp.maximum(c, 1.0)
    zt = jnp.concatenate([mean, mn, mx], axis=0)
    out_ref[...] = (
        jax.lax.dot_general(zt, wt_ref[...], (((0,), (0,)), ((), ())),
                            preferred_element_type=jnp.float32)
        + b_ref[...]
    )


def _merge_tc(ps, pm, px, pc, wt, b2):
    return pl.pallas_call(
        _merge_tc_body,
        out_shape=jax.ShapeDtypeStruct((NG, DG), jnp.float32),
    )(ps, pm, px, pc, wt, b2)


@jax.jit
def kernel(fe, segment_ids, W, b):
    ids = segment_ids.astype(jnp.int32)
    fe2 = fe.reshape(_NT, _TILE, 2, 8).transpose(2, 0, 3, 1).reshape(
        2, _HALF)
    ps, pm, px, pc = _seg_reduce_sc(fe2, ids)
    nw = ps.shape[0] // DE
    return _merge_tc(
        ps.reshape(nw, DE, NG), pm.reshape(nw, DE, NG),
        px.reshape(nw, DE, NG), pc.reshape(nw, DE, NG), W.T, b[None, :])


# speedup vs baseline: 1.0143x; 1.0143x over previous
import functools

import jax
import jax.numpy as jnp
from jax import lax
from jax.experimental import pallas as pl
from jax.experimental.pallas import tpu as pltpu
from jax.experimental.pallas import tpu_sc as plsc

E = 3_200_000
DE = 16
DG = 128
NG = 256

_TILE = 128
_NT = E // _TILE
_CT = 16
_CE = _CT * _TILE
_HALF = _NT * 8 * _TILE


def _seg_reduce_sc(fe2, seg_ids):
    info = plsc.get_sparse_core_info()
    nc, ns = info.num_cores, info.num_subcores
    nw = nc * ns
    base_t = _NT // nw
    rem_t = _NT % nw
    n_chunks = base_t // _CT + 1
    mesh = plsc.VectorSubcoreMesh(core_axis_name="c", subcore_axis_name="s")

    neg_inf = jnp.float32(-jnp.inf)
    pos_inf = jnp.float32(jnp.inf)

    @functools.partial(
        pl.kernel,
        mesh=mesh,
        compiler_params=pltpu.CompilerParams(
            use_tc_tiling_on_sc=False, needs_layout_passes=False),
        out_type=[
            jax.ShapeDtypeStruct((nw * DE, NG), jnp.float32),
            jax.ShapeDtypeStruct((nw * DE, NG), jnp.float32),
            jax.ShapeDtypeStruct((nw * DE, NG), jnp.float32),
            jax.ShapeDtypeStruct((nw * DE, NG), jnp.float32),
        ],
        scratch_types=[
            pltpu.VMEM((2 * _CE * 8,), jnp.float32),
            pltpu.VMEM((2 * _CE * 8,), jnp.float32),
            pltpu.VMEM((_CE,), jnp.int32),
            pltpu.VMEM((_CE,), jnp.int32),
            pltpu.VMEM((DE, NG), jnp.float32),
            pltpu.VMEM((DE, NG), jnp.float32),
            pltpu.VMEM((DE, NG), jnp.float32),
            pltpu.VMEM((DE, NG), jnp.float32),
            pltpu.VMEM((DE * 16,), jnp.float32),
            pltpu.VMEM((DE * 16,), jnp.float32),
            pltpu.VMEM((DE * 16,), jnp.float32),
            pltpu.VMEM((DE,), jnp.float32),
            pltpu.VMEM((DE,), jnp.float32),
            pltpu.VMEM((DE,), jnp.float32),
            pltpu.SMEM((1,), jnp.int32),
            pltpu.SMEM((1,), jnp.float32),
            pltpu.SemaphoreType.DMA,
            pltpu.SemaphoreType.DMA,
            pltpu.SemaphoreType.DMA,
            pltpu.SemaphoreType.DMA,
        ],
    )
    def k(fe_hbm, ids_hbm, o_sum, o_min, o_max, o_cnt,
          fe_v0, fe_v1, ids_v0, ids_v1, sum_t, min_t, max_t, cnt_t,
          es_s, es_mn, es_mx, fs_s, fs_mn, fs_mx, cur_ref, cnt_ref,
          fsem0, fsem1, isem0, isem1):
        wid = lax.axis_index("c") * ns + lax.axis_index("s")
        tb = wid * base_t + jnp.minimum(wid, rem_t)
        ntiles = base_t + jnp.where(wid < rem_t, 1, 0)
        tail_t0 = tb + ntiles - _CT
        tail_skip = _CT - (ntiles - (n_chunks - 1) * _CT)

        fe_bufs = (fe_v0, fe_v1)
        ids_bufs = (ids_v0, ids_v1)
        fsems = (fsem0, fsem1)
        isems = (isem0, isem1)

        dvec = lax.iota(jnp.int32, 16)
        featoff = (dvec >> 3) * (_CE * 8) + (dvec & 7) * _TILE

        def tile0_of(c):
            return jnp.where(c < n_chunks - 1, tb + c * _CT, tail_t0)

        def fe_dma(c, b, half):
            start = tile0_of(c) * (8 * _TILE)
            return pltpu.make_async_copy(
                fe_hbm.at[half, pl.ds(start, _CE * 8)],
                fe_bufs[b].at[pl.ds(half * (_CE * 8), _CE * 8)], fsems[b])

        def ids_dma(c, b):
            start = tile0_of(c) * _TILE
            return pltpu.make_async_copy(
                ids_hbm.at[pl.ds(start, _CE)], ids_bufs[b], isems[b])

        def init_col(j, _):
            z16 = jnp.zeros((16,), jnp.float32)
            p16 = jnp.full((16,), pos_inf, jnp.float32)
            n16 = jnp.full((16,), neg_inf, jnp.float32)
            for d in range(DE):
                sum_t[d, pl.ds(j * 16, 16)] = z16
                min_t[d, pl.ds(j * 16, 16)] = p16
                max_t[d, pl.ds(j * 16, 16)] = n16
                cnt_t[d, pl.ds(j * 16, 16)] = z16
            return 0

        lax.fori_loop(0, NG // 16, init_col, 0)

        def reset_accs():
            for d in range(DE):
                es_s[pl.ds(d * 16, 16)] = jnp.zeros((16,), jnp.float32)
                es_mn[pl.ds(d * 16, 16)] = jnp.full((16,), pos_inf,
                                                    jnp.float32)
                es_mx[pl.ds(d * 16, 16)] = jnp.full((16,), neg_inf,
                                                    jnp.float32)
            fs_s[...] = jnp.zeros((DE,), jnp.float32)
            fs_mn[...] = jnp.full((DE,), pos_inf, jnp.float32)
            fs_mx[...] = jnp.full((DE,), neg_inf, jnp.float32)
            cnt_ref[0] = jnp.float32(0.0)

        reset_accs()
        cur_ref[0] = jnp.int32(-1)

        def flush():
            cur = cur_ref[0]

            @pl.when(cur >= 0)
            def _():
                row_s = fs_s[...]
                row_mn = fs_mn[...]
                row_mx = fs_mx[...]
                for d in range(DE):
                    oh = dvec == d
                    vs = jnp.sum(es_s[pl.ds(d * 16, 16)])
                    vmn = jnp.min(es_mn[pl.ds(d * 16, 16)])
                    vmx = jnp.max(es_mx[pl.ds(d * 16, 16)])
                    row_s = row_s + jnp.where(
                        oh, jax.lax.broadcast(vs, (16,)), 0.0)
                    row_mn = jnp.minimum(row_mn, jnp.where(
                        oh, jax.lax.broadcast(vmn, (16,)), pos_inf))
                    row_mx = jnp.maximum(row_mx, jnp.where(
                        oh, jax.lax.broadcast(vmx, (16,)), neg_inf))
                curv = jax.lax.broadcast(cur, (16,))
                old_s = plsc.load_gather(sum_t, [dvec, curv])
                plsc.store_scatter(sum_t, [dvec, curv], old_s + row_s)
                old_mn = plsc.load_gather(min_t, [dvec, curv])
                plsc.store_scatter(min_t, [dvec, curv],
                                   jnp.minimum(old_mn, row_mn))
                old_mx = plsc.load_gather(max_t, [dvec, curv])
                plsc.store_scatter(max_t, [dvec, curv],
                                   jnp.maximum(old_mx, row_mx))
                old_c = plsc.load_gather(cnt_t, [dvec, curv])
                plsc.store_scatter(cnt_t, [dvec, curv],
                                   old_c + jnp.full((16,), cnt_ref[0]))
                reset_accs()

        def process_chunk(fe_b, ids_b, t_lo):
            g_lo = t_lo * 8
            c_first = ids_b[pl.ds(t_lo * _TILE, 16)][0]
            c_last = ids_b[pl.ds(_CE - 16, 16)][15]
            ccur = cur_ref[0]
            cfast = jnp.logical_and(c_first == ccur, c_last == ccur)

            @pl.when(cfast)
            def _():
                for pf in range(2):
                    accs = []
                    for j in range(8):
                        o = (pf * 8 + j) * 16
                        accs.append(es_s[pl.ds(o, 16)])
                        accs.append(es_mn[pl.ds(o, 16)])
                        accs.append(es_mx[pl.ds(o, 16)])

                    def pass_body(g, carry, pf=pf):
                        base = (g >> 3) * (8 * _TILE) + (g & 7) * 16
                        out = list(carry)
                        for j in range(8):
                            off = (pf * (_CE * 8) + j * _TILE) + base
                            v = fe_b[pl.ds(off, 16)]
                            out[3 * j] = out[3 * j] + v
                            out[3 * j + 1] = jnp.minimum(out[3 * j + 1], v)
                            out[3 * j + 2] = jnp.maximum(out[3 * j + 2], v)
                        return tuple(out)

                    accs = lax.fori_loop(g_lo, _CE // 16, pass_body,
                                         tuple(accs))
                    for j in range(8):
                        o = (pf * 8 + j) * 16
                        es_s[pl.ds(o, 16)] = accs[3 * j]
                        es_mn[pl.ds(o, 16)] = accs[3 * j + 1]
                        es_mx[pl.ds(o, 16)] = accs[3 * j + 2]
                cnt_ref[0] = cnt_ref[0] + (
                    jnp.float32(_CE) - jnp.float32(16.0) * g_lo.astype(
                        jnp.float32))

            @pl.when(jnp.logical_not(cfast))
            def _():
                _process_tiles(fe_b, ids_b, t_lo)

        def _process_tiles(fe_b, ids_b, t_lo):
            def tile_body(t, _):
                e0 = t * _TILE
                id_a = ids_b[pl.ds(e0, 16)][0]
                id_b = ids_b[pl.ds(e0 + 112, 16)][15]
                cur = cur_ref[0]
                tfast = jnp.logical_and(id_a == cur, id_b == cur)

                @pl.when(tfast)
                def _():
                    for pf in range(2):
                        accs = []
                        for j in range(8):
                            d = pf * 8 + j
                            o = d * 16
                            accs.append([es_s[pl.ds(o, 16)],
                                         es_mn[pl.ds(o, 16)],
                                         es_mx[pl.ds(o, 16)]])
                        for g in range(8):
                            for j in range(8):
                                d = pf * 8 + j
                                off = ((d >> 3) * (_CE * 8) + (d & 7) * _TILE
                                       + g * 16) + t * (8 * _TILE)
                                v = fe_b[pl.ds(off, 16)]
                                a = accs[j]
                                a[0] = a[0] + v
                                a[1] = jnp.minimum(a[1], v)
                                a[2] = jnp.maximum(a[2], v)
                        for j in range(8):
                            d = pf * 8 + j
                            o = d * 16
                            es_s[pl.ds(o, 16)] = accs[j][0]
                            es_mn[pl.ds(o, 16)] = accs[j][1]
                            es_mx[pl.ds(o, 16)] = accs[j][2]
                    cnt_ref[0] = cnt_ref[0] + jnp.float32(_TILE)

                @pl.when(jnp.logical_not(tfast))
                def _():
                    def group_body(g, _g):
                        idv = ids_b[pl.ds(e0 + g * 16, 16)]
                        curg = cur_ref[0]
                        gfast = jnp.logical_and(idv[0] == curg,
                                                idv[15] == curg)

                        @pl.when(gfast)
                        def _():
                            for d in range(DE):
                                off = ((d >> 3) * (_CE * 8)
                                       + (d & 7) * _TILE) + t * (8 * _TILE)
                                v = fe_b[pl.ds(off + g * 16, 16)]
                                o = d * 16
                                es_s[pl.ds(o, 16)] = es_s[pl.ds(o, 16)] + v
                                es_mn[pl.ds(o, 16)] = jnp.minimum(
                                    es_mn[pl.ds(o, 16)], v)
                                es_mx[pl.ds(o, 16)] = jnp.maximum(
                                    es_mx[pl.ds(o, 16)], v)
                            cnt_ref[0] = cnt_ref[0] + jnp.float32(16.0)

                        @pl.when(jnp.logical_not(gfast))
                        def _():
                            for l in range(16):
                                eid = idv[l]

                                @pl.when(eid != cur_ref[0])
                                def _(eid=eid):
                                    flush()
                                    cur_ref[0] = eid

                                idx = featoff + (t * (8 * _TILE)
                                                 + g * 16 + l)
                                row = plsc.load_gather(fe_b, [idx])
                                fs_s[...] = fs_s[...] + row
                                fs_mn[...] = jnp.minimum(fs_mn[...], row)
                                fs_mx[...] = jnp.maximum(fs_mx[...], row)
                                cnt_ref[0] = cnt_ref[0] + 1.0

                        return 0

                    lax.fori_loop(0, 8, group_body, 0)

                return 0

            lax.fori_loop(t_lo, _CT, tile_body, 0)

        fe_dma(0, 0, 0).start()
        fe_dma(0, 0, 1).start()
        ids_dma(0, 0).start()

        def pair_body(i, _):
            for b in range(2):
                c = i * 2 + b

                @pl.when(c < n_chunks)
                def _(c=c, b=b):
                    @pl.when(c + 1 < n_chunks)
                    def _():
                        fe_dma(c + 1, 1 - b, 0).start()
                        fe_dma(c + 1, 1 - b, 1).start()
                        ids_dma(c + 1, 1 - b).start()

                    fe_dma(c, b, 0).wait()
                    fe_dma(c, b, 1).wait()
                    ids_dma(c, b).wait()
                    t_lo = jnp.where(c < n_chunks - 1, 0, tail_skip)
                    process_chunk(fe_bufs[b], ids_bufs[b], t_lo)
            return 0

        lax.fori_loop(0, (n_chunks + 1) // 2, pair_body, 0)
        flush()

        pltpu.sync_copy(sum_t, o_sum.at[pl.ds(wid * DE, DE), :])
        pltpu.sync_copy(min_t, o_min.at[pl.ds(wid * DE, DE), :])
        pltpu.sync_copy(max_t, o_max.at[pl.ds(wid * DE, DE), :])
        pltpu.sync_copy(cnt_t, o_cnt.at[pl.ds(wid * DE, DE), :])

    return k(fe2, seg_ids)


def _merge_tc_body(ps_ref, pm_ref, px_ref, pc_ref, wt_ref, b_ref, out_ref):
    s = jnp.sum(ps_ref[...], axis=0)
    mn = jnp.min(pm_ref[...], axis=0)
    mx = jnp.max(px_ref[...], axis=0)
    c = jnp.sum(pc_ref[...], axis=0)
    mean = s / jnp.maximum(c, 1.0)
    zt = jnp.concatenate([mean, mn, mx], axis=0)
    out_ref[...] = (
        jax.lax.dot_general(zt, wt_ref[...], (((0,), (0,)), ((), ())),
                            preferred_element_type=jnp.float32)
        + b_ref[...]
    )


def _merge_tc(ps, pm, px, pc, wt, b2):
    return pl.pallas_call(
        _merge_tc_body,
        out_shape=jax.ShapeDtypeStruct((NG, DG), jnp.float32),
    )(ps, pm, px, pc, wt, b2)


@jax.jit
def kernel(fe, segment_ids, W, b):
    ids = segment_ids.astype(jnp.int32)
    fe2 = fe.reshape(_NT, _TILE, 2, 8).transpose(2, 0, 3, 1).reshape(
        2, _HALF)
    ps, pm, px, pc = _seg_reduce_sc(fe2, ids)
    nw = ps.shape[0] // DE
    return _merge_tc(
        ps.reshape(nw, DE, NG), pm.reshape(nw, DE, NG),
        px.reshape(nw, DE, NG), pc.reshape(nw, DE, NG), W.T, b[None, :])
